# baseline (speedup 1.0000x reference)
import functools

import jax
import jax.numpy as jnp
from jax import lax
from jax.experimental import pallas as pl
from jax.experimental.pallas import tpu as pltpu, tpu_sc as plsc

DICT = 100
B, T = 1024, 200
NBIN = 128
WSC = 224
WOUT = 205
NW = 32
RPW = B // NW
EPS = 1e-09


def _sc_body(amt_hbm, mcc_hbm, out_hbm, amt_v, mcc_v, out_v, hc, hs):
    wid = lax.axis_index("s") * 2 + lax.axis_index("c")
    base = wid * RPW

    pltpu.sync_copy(amt_hbm.at[pl.ds(base, RPW)], amt_v)
    pltpu.sync_copy(mcc_hbm.at[pl.ds(base, RPW)], mcc_v)

    iota = lax.iota(jnp.int32, 16)
    zero = jnp.zeros((16,), jnp.float32)
    ones = jnp.ones((16,), jnp.float32)
    tail_keep = iota >= 8

    def row_work(r, _):
        for k in range(7):
            hc[pl.ds(k * 16, 16)] = zero
            hs[pl.ds(k * 16, 16)] = zero

        vals = []
        idxs = []
        cidxs = []
        for j in range(13):
            off = j * 16 if j < 12 else 184
            a = amt_v[r, pl.ds(off, 16)]
            v = jnp.sign(a) * (jnp.exp(jnp.abs(a)) - 1.0)
            idx = jnp.clip(mcc_v[r, pl.ds(off, 16)], 0, DICT - 1)
            cidx = idx
            if j == 12:
                v = jnp.where(tail_keep, v, 0.0)
                cidx = jnp.where(tail_keep, idx, 0)
            vals.append(v)
            idxs.append(idx)
            cidxs.append(cidx)
        for j in range(13):
            plsc.addupdate_scatter(hc, [cidxs[j]], ones)
            plsc.addupdate_scatter(hs, [idxs[j]], vals[j])

        rb = r * WSC
        dcnt = zero
        for k in range(7):
            c = hc[pl.ds(k * 16, 16)]
            s = hs[pl.ds(k * 16, 16)]
            if k == 0:
                c = jnp.where(iota == 0, 0.0, c)
            em = s / (c + 1e-09)
            out_v[pl.ds(rb + k * 16, 16)] = c
            dcnt = dcnt + jnp.where(c > 0.0, 1.0, 0.0)
            if k < 6:
                out_v[pl.ds(rb + 112 + k * 16, 16)] = em
            else:
                em = jnp.where(iota == 4, jnp.sum(dcnt), em)
                out_v[pl.ds(rb + 112 + k * 16, 16)] = em
        return 0

    lax.fori_loop(0, RPW, row_work, 0)
    pltpu.sync_copy(out_v, out_hbm.at[pl.ds(base * WSC, RPW * WSC)])


def _tc_body(amt_ref, sl_ref, sc_ref, out_ref):
    a = amt_ref[...]
    val = jnp.sign(a) * (jnp.exp(jnp.abs(a)) - 1.0)
    sum_ = jnp.sum(val, axis=1, keepdims=True)
    sumsq = jnp.sum(val * val, axis=1, keepdims=True)
    slf = sl_ref[0].reshape(-1, 1).astype(jnp.float32)
    mean = sum_ / (slf + EPS)
    var_num = jnp.maximum(sumsq - sum_ * sum_ / (slf + EPS), 0.0)
    std = jnp.sqrt(var_num / (jnp.maximum(slf - 1.0, 0.0) + EPS))
    sc = sc_ref[...]
    out_ref[...] = jnp.concatenate(
        [slf, sum_, mean, std, sc[:, 0:100], sc[:, 112:212], sc[:, 212:213]],
        axis=1)


@jax.jit
def _run(amount, mcc, seq_lens):
    mesh = plsc.VectorSubcoreMesh(core_axis_name="c", subcore_axis_name="s")
    sc_hist = functools.partial(
        pl.kernel,
        out_type=jax.ShapeDtypeStruct((B * WSC,), jnp.float32),
        mesh=mesh,
        scratch_types=[
            pltpu.VMEM((RPW, T), jnp.float32),
            pltpu.VMEM((RPW, T), jnp.int32),
            pltpu.VMEM((RPW * WSC,), jnp.float32),
            pltpu.VMEM((NBIN,), jnp.float32),
            pltpu.VMEM((NBIN,), jnp.float32),
        ],
        compiler_params=pltpu.CompilerParams(needs_layout_passes=False),
    )(_sc_body)
    sc_out = sc_hist(amount, mcc).reshape(B, WSC)

    rows_blk = 128
    out = pl.pallas_call(
        _tc_body,
        out_shape=jax.ShapeDtypeStruct((B, WOUT), jnp.float32),
        grid=(B // rows_blk,),
        in_specs=[
            pl.BlockSpec((rows_blk, T), lambda i: (i, 0)),
            pl.BlockSpec((1, 1, rows_blk), lambda i: (i, 0, 0)),
            pl.BlockSpec((rows_blk, WSC), lambda i: (i, 0)),
        ],
        out_specs=pl.BlockSpec((rows_blk, WOUT), lambda i: (i, 0)),
    )(amount, seq_lens.reshape(B // rows_blk, 1, rows_blk), sc_out)
    return out


def kernel(amount, mcc, seq_lens):
    return _run(amount, mcc.astype(jnp.int32), seq_lens.astype(jnp.int32))


# speedup vs baseline: 1.1517x; 1.1517x over previous
import functools

import jax
import jax.numpy as jnp
from jax import lax
from jax.experimental import pallas as pl
from jax.experimental.pallas import tpu as pltpu, tpu_sc as plsc

DICT = 100
B, T = 1024, 200
NBIN = 128
W = 240
NW = 32
RPW = B // NW
EPS = 1e-09


def _body(amt_hbm, mcc_hbm, sl_hbm, out_hbm, amt_v, mcc_v, sl_v, out_v, hc, hs):
    wid = lax.axis_index("s") * 2 + lax.axis_index("c")
    base = wid * RPW

    pltpu.sync_copy(amt_hbm.at[pl.ds(base, RPW)], amt_v)
    pltpu.sync_copy(mcc_hbm.at[pl.ds(base, RPW)], mcc_v)
    pltpu.sync_copy(sl_hbm.at[pl.ds(base, RPW)], sl_v.at[pl.ds(0, RPW)])

    iota = lax.iota(jnp.int32, 16)
    zero = jnp.zeros((16,), jnp.float32)
    ones = jnp.ones((16,), jnp.float32)
    tail_keep = iota >= 8

    def row_work(r, _):
        for k in range(7):
            hc[pl.ds(k * 16, 16)] = zero
            hs[pl.ds(k * 16, 16)] = zero

        acc_s = zero
        acc_q = zero
        vals = []
        idxs = []
        cidxs = []
        for j in range(13):
            off = j * 16 if j < 12 else 184
            a = amt_v[r, pl.ds(off, 16)]
            v = jnp.sign(a) * (jnp.exp(jnp.abs(a)) - 1.0)
            idx = jnp.clip(mcc_v[r, pl.ds(off, 16)], 0, DICT - 1)
            cidx = idx
            if j == 12:
                v = jnp.where(tail_keep, v, 0.0)
                cidx = jnp.where(tail_keep, idx, 0)
            vals.append(v)
            idxs.append(idx)
            cidxs.append(cidx)
            acc_s = acc_s + v
            acc_q = acc_q + v * v
        for j in range(13):
            plsc.addupdate_scatter(hc, [cidxs[j]], ones)
            plsc.addupdate_scatter(hs, [idxs[j]], vals[j])

        sum_ = jnp.full((16,), jnp.sum(acc_s))
        sumsq = jnp.full((16,), jnp.sum(acc_q))

        slf = jnp.full((16,), sl_v[pl.ds(r, 16)][0].astype(jnp.float32))
        mean = sum_ / (slf + EPS)
        var_num = jnp.maximum(sumsq - sum_ * sum_ / (slf + EPS), 0.0)
        var = var_num / (jnp.maximum(slf - 1.0, 0.0) + EPS)

        dcnt = zero
        for k in range(7):
            c = hc[pl.ds(k * 16, 16)]
            s = hs[pl.ds(k * 16, 16)]
            if k == 0:
                c = jnp.where(iota == 0, 0.0, c)
            em = s / (c + 1e-09)
            out_v[r, pl.ds(16 + k * 16, 16)] = c
            dcnt = dcnt + jnp.where(c > 0.0, 1.0, 0.0)
            if k < 6:
                out_v[r, pl.ds(128 + k * 16, 16)] = em
            else:
                em = jnp.where(iota == 4, jnp.sum(dcnt), em)
                out_v[r, pl.ds(128 + k * 16, 16)] = em

        x = jnp.where(iota == 3, var, 1.0)
        bits = lax.bitcast_convert_type(x, jnp.int32)
        y = lax.bitcast_convert_type(
            lax.shift_right_arithmetic(bits, 1) + jnp.int32(0x1FBD1DF5),
            jnp.float32)
        for _ in range(4):
            y = 0.5 * (y + x / y)

        head = jnp.where(iota == 0, slf,
               jnp.where(iota == 1, sum_,
               jnp.where(iota == 2, mean,
               jnp.where(iota == 3, y, 0.0))))
        out_v[r, pl.ds(0, 16)] = head
        return 0

    lax.fori_loop(0, RPW, row_work, 0)
    pltpu.sync_copy(out_v, out_hbm.at[pl.ds(base, RPW)])


@jax.jit
def _run(amount, mcc, seq_lens):
    mesh = plsc.VectorSubcoreMesh(core_axis_name="c", subcore_axis_name="s")
    k = functools.partial(
        pl.kernel,
        out_type=jax.ShapeDtypeStruct((B, W), jnp.float32),
        mesh=mesh,
        scratch_types=[
            pltpu.VMEM((RPW, T), jnp.float32),
            pltpu.VMEM((RPW, T), jnp.int32),
            pltpu.VMEM((RPW + 16,), jnp.int32),
            pltpu.VMEM((RPW, W), jnp.float32),
            pltpu.VMEM((NBIN,), jnp.float32),
            pltpu.VMEM((NBIN,), jnp.float32),
        ],
        compiler_params=pltpu.CompilerParams(needs_layout_passes=False),
    )(_body)
    return k(amount, mcc, seq_lens)


def kernel(amount, mcc, seq_lens):
    out = _run(amount, mcc.astype(jnp.int32), seq_lens.astype(jnp.int32))
    return jnp.concatenate(
        [out[:, 0:4], out[:, 16:116], out[:, 128:228], out[:, 228:229]], axis=1)


# speedup vs baseline: 1.1799x; 1.0245x over previous
import functools

import jax
import jax.numpy as jnp
from jax import lax
from jax.experimental import pallas as pl
from jax.experimental.pallas import tpu as pltpu, tpu_sc as plsc

DICT = 100
B, T = 1024, 200
NBIN = 128
W = 240
NW = 32
RPW = B // NW
EPS = 1e-09


def _body(amt_hbm, mcc_hbm, sl_hbm, out_hbm, amt_v, mcc_v, sl_v, out_v, hc, hs):
    wid = lax.axis_index("s") * 2 + lax.axis_index("c")
    base = wid * RPW

    pltpu.sync_copy(amt_hbm.at[pl.ds(base, RPW)], amt_v)
    pltpu.sync_copy(mcc_hbm.at[pl.ds(base, RPW)], mcc_v)
    pltpu.sync_copy(sl_hbm.at[pl.ds(base, RPW)], sl_v.at[pl.ds(0, RPW)])

    iota = lax.iota(jnp.int32, 16)
    zero = jnp.zeros((16,), jnp.float32)
    ones = jnp.ones((16,), jnp.float32)
    tail_keep = iota >= 8

    def row_work(r, _):
        for k in range(7):
            hc[pl.ds(k * 16, 16)] = zero
            hs[pl.ds(k * 16, 16)] = zero

        acc_s = zero
        acc_q = zero
        vals = []
        idxs = []
        cidxs = []
        for j in range(13):
            off = j * 16 if j < 12 else 184
            a = amt_v[r, pl.ds(off, 16)]
            v = jnp.sign(a) * (jnp.exp(jnp.abs(a)) - 1.0)
            idx = jnp.clip(mcc_v[r, pl.ds(off, 16)], 0, DICT - 1)
            cidx = idx
            if j == 12:
                v = jnp.where(tail_keep, v, 0.0)
                cidx = jnp.where(tail_keep, idx, 0)
            vals.append(v)
            idxs.append(idx)
            cidxs.append(cidx)
            acc_s = acc_s + v
            acc_q = acc_q + v * v
        for j in range(13):
            plsc.addupdate_scatter(hc, [cidxs[j]], ones)
            plsc.addupdate_scatter(hs, [idxs[j]], vals[j])

        sum_ = jnp.full((16,), jnp.sum(acc_s))
        sumsq = jnp.full((16,), jnp.sum(acc_q))

        slf = jnp.full((16,), sl_v[pl.ds(r, 16)][0].astype(jnp.float32))
        mean = sum_ / (slf + EPS)
        var_num = jnp.maximum(sumsq - sum_ * sum_ / (slf + EPS), 0.0)
        var = var_num / (jnp.maximum(slf - 1.0, 0.0) + EPS)

        rb = r * 205
        dcnt = zero
        for k in range(7):
            c = hc[pl.ds(k * 16, 16)]
            s = hs[pl.ds(k * 16, 16)]
            if k == 0:
                c = jnp.where(iota == 0, 0.0, c)
            em = s / (c + 1e-09)
            dcnt = dcnt + jnp.where(c > 0.0, 1.0, 0.0)
            if k < 6:
                plsc.store_scatter(out_v, [rb + 4 + k * 16 + iota], c)
                plsc.store_scatter(out_v, [rb + 104 + k * 16 + iota], em)
            else:
                plsc.store_scatter(out_v, [rb + 100 + iota], c, mask=iota < 4)
                em = jnp.where(iota == 4, jnp.sum(dcnt), em)
                plsc.store_scatter(out_v, [rb + 200 + iota], em, mask=iota < 5)

        x = jnp.where(iota == 3, var, 1.0)
        bits = lax.bitcast_convert_type(x, jnp.int32)
        y = lax.bitcast_convert_type(
            lax.shift_right_arithmetic(bits, 1) + jnp.int32(0x1FBD1DF5),
            jnp.float32)
        for _ in range(4):
            y = 0.5 * (y + x / y)

        head = jnp.where(iota == 0, slf,
               jnp.where(iota == 1, sum_,
               jnp.where(iota == 2, mean,
               jnp.where(iota == 3, y, 0.0))))
        plsc.store_scatter(out_v, [rb + iota], head, mask=iota < 4)
        return 0

    lax.fori_loop(0, RPW, row_work, 0)
    pltpu.sync_copy(out_v, out_hbm.at[pl.ds(base * 205, RPW * 205)])


@jax.jit
def _run(amount, mcc, seq_lens):
    mesh = plsc.VectorSubcoreMesh(core_axis_name="c", subcore_axis_name="s")
    k = functools.partial(
        pl.kernel,
        out_type=jax.ShapeDtypeStruct((B * 205,), jnp.float32),
        mesh=mesh,
        scratch_types=[
            pltpu.VMEM((RPW, T), jnp.float32),
            pltpu.VMEM((RPW, T), jnp.int32),
            pltpu.VMEM((RPW + 16,), jnp.int32),
            pltpu.VMEM((RPW * 205,), jnp.float32),
            pltpu.VMEM((NBIN,), jnp.float32),
            pltpu.VMEM((NBIN,), jnp.float32),
        ],
        compiler_params=pltpu.CompilerParams(needs_layout_passes=False),
    )(_body)
    return k(amount, mcc, seq_lens)


def kernel(amount, mcc, seq_lens):
    out = _run(amount, mcc.astype(jnp.int32), seq_lens.astype(jnp.int32))
    return out.reshape(B, 205)
